# baseline (speedup 1.0000x reference)
import functools

import jax
import jax.numpy as jnp
from jax import lax
from jax.experimental import pallas as pl
from jax.experimental.pallas import tpu as pltpu
from jax.experimental.pallas import tpu_sc as plsc

_D = 64
_B = 16384
_NC = 2
_NS = 16
_NW = _NC * _NS
_BPW = _B // _NW
_L = 16
_CH = 64
_NCHK = _BPW // _CH

_mesh = plsc.VectorSubcoreMesh(core_axis_name="c", subcore_axis_name="s")


@functools.partial(
    pl.kernel,
    mesh=_mesh,
    compiler_params=pltpu.CompilerParams(use_tc_tiling_on_sc=False,
                                         needs_layout_passes=False),
    out_type=[jax.ShapeDtypeStruct((_D, _B), jnp.float32)] * 3,
    scratch_types=[
        pltpu.VMEM((_BPW,), jnp.int32),
        pltpu.VMEM((_BPW,), jnp.int32),
        pltpu.VMEM((_BPW,), jnp.int32),
        pltpu.VMEM((_D, _CH * 8), jnp.float32),
        pltpu.VMEM((_D, _CH * 8), jnp.float32),
        pltpu.VMEM((_D, _CH * 8), jnp.float32),
        pltpu.VMEM((_D, _CH), jnp.float32),
        pltpu.VMEM((_D, _CH), jnp.float32),
        pltpu.VMEM((_D, _CH), jnp.float32),
        pltpu.SemaphoreType.DMA,
    ],
)
def _gather3(h_idx, r_idx, t_idx, node_t, rel_t,
             h_out, r_out, t_out,
             h_ix, r_ix, t_ix, h_oct, r_oct, t_oct, h_col, r_col, t_col,
             sem):
    wid = lax.axis_index("s") * _NC + lax.axis_index("c")
    base = wid * _BPW
    pltpu.sync_copy(h_idx.at[pl.ds(base, _BPW)], h_ix)
    pltpu.sync_copy(r_idx.at[pl.ds(base, _BPW)], r_ix)
    pltpu.sync_copy(t_idx.at[pl.ds(base, _BPW)], t_ix)

    rows16 = lax.iota(jnp.int32, _L)

    def do_chunk(c, carry):
        cb = c * _CH

        def issue16(q, carry2):
            b16 = cb + q * _L
            hv = h_ix[pl.ds(b16, _L)]
            rv = r_ix[pl.ds(b16, _L)]
            tv = t_ix[pl.ds(b16, _L)]
            for k in range(_L):
                j = (q * _L + k) * 8
                ha = pl.multiple_of(lax.bitwise_and(hv[k], jnp.int32(~7)), 8)
                ra = pl.multiple_of(lax.bitwise_and(rv[k], jnp.int32(~7)), 8)
                ta = pl.multiple_of(lax.bitwise_and(tv[k], jnp.int32(~7)), 8)
                pltpu.async_copy(node_t.at[:, pl.ds(ha, 8)],
                                 h_oct.at[:, pl.ds(j, 8)], sem)
                pltpu.async_copy(rel_t.at[:, pl.ds(ra, 8)],
                                 r_oct.at[:, pl.ds(j, 8)], sem)
                pltpu.async_copy(node_t.at[:, pl.ds(ta, 8)],
                                 t_oct.at[:, pl.ds(j, 8)], sem)
            return carry2

        lax.fori_loop(0, _CH // _L, issue16, 0)

        def drain1(i, carry2):
            pltpu.make_async_copy(node_t.at[:, pl.ds(0, 8)],
                                  h_oct.at[:, pl.ds(i * 8, 8)], sem).wait()
            pltpu.make_async_copy(rel_t.at[:, pl.ds(0, 8)],
                                  r_oct.at[:, pl.ds(i * 8, 8)], sem).wait()
            pltpu.make_async_copy(node_t.at[:, pl.ds(0, 8)],
                                  t_oct.at[:, pl.ds(i * 8, 8)], sem).wait()
            return carry2

        lax.fori_loop(0, _CH, drain1, 0)

        def extract16(q, carry2):
            b16 = cb + q * _L
            hv = h_ix[pl.ds(b16, _L)]
            rv = r_ix[pl.ds(b16, _L)]
            tv = t_ix[pl.ds(b16, _L)]
            for k in range(_L):
                j = q * _L + k
                hc = jnp.broadcast_to(j * 8 + lax.bitwise_and(hv[k], jnp.int32(7)), (_L,))
                rc = jnp.broadcast_to(j * 8 + lax.bitwise_and(rv[k], jnp.int32(7)), (_L,))
                tc = jnp.broadcast_to(j * 8 + lax.bitwise_and(tv[k], jnp.int32(7)), (_L,))
                jv = jnp.broadcast_to(jnp.int32(j), (_L,))
                for q2 in range(_D // _L):
                    rr = rows16 + q2 * _L
                    plsc.store_scatter(h_col, [rr, jv],
                                       plsc.load_gather(h_oct, [rr, hc]))
                    plsc.store_scatter(r_col, [rr, jv],
                                       plsc.load_gather(r_oct, [rr, rc]))
                    plsc.store_scatter(t_col, [rr, jv],
                                       plsc.load_gather(t_oct, [rr, tc]))
            return carry2

        lax.fori_loop(0, _CH // _L, extract16, 0)

        dst = pl.ds(base + cb, _CH)
        pltpu.sync_copy(h_col, h_out.at[:, dst])
        pltpu.sync_copy(r_col, r_out.at[:, dst])
        pltpu.sync_copy(t_col, t_out.at[:, dst])
        return carry

    lax.fori_loop(0, _NCHK, do_chunk, 0)


def kernel(triplets, node_embs, rel_embs):
    tri = triplets.astype(jnp.int32)
    h_idx = tri[:, 0].reshape(_B)
    r_idx = tri[:, 1].reshape(_B)
    t_idx = tri[:, 2].reshape(_B)
    h_t, r_t, t_t = _gather3(h_idx, r_idx, t_idx, node_embs.T, rel_embs.T)
    return (h_t.T, r_t.T, t_t.T)


# speedup vs baseline: 42.8156x; 42.8156x over previous
import functools

import jax
import jax.numpy as jnp
from jax import lax
from jax.experimental import pallas as pl
from jax.experimental.pallas import tpu as pltpu
from jax.experimental.pallas import tpu_sc as plsc

_D = 64
_B = 16384
_NC = 2
_NS = 16
_NW = _NC * _NS
_HN = 2 * _B // _NW
_HR = _B // _NW
_L = 16
_BW = 128
_R = 8
_NST = 16

_mesh = plsc.VectorSubcoreMesh(core_axis_name="c", subcore_axis_name="s")


def _splat(x):
    return jnp.broadcast_to(x.astype(jnp.int32), (_L,))


@functools.partial(
    pl.kernel,
    mesh=_mesh,
    compiler_params=pltpu.CompilerParams(disable_bounds_checks=True,
                                         needs_layout_passes=False),
    out_type=[jax.ShapeDtypeStruct((_B, _D), jnp.float32)] * 3,
    scratch_types=[
        pltpu.VMEM((_HN,), jnp.int32),
        pltpu.VMEM((_HN,), jnp.int32),
        pltpu.VMEM((_HR,), jnp.int32),
        pltpu.VMEM((_HR,), jnp.int32),
        pltpu.VMEM((_R, _D, _BW), jnp.float32),
        pltpu.VMEM((_NST, _D), jnp.float32),
        pltpu.SemaphoreType.DMA,
        pltpu.SemaphoreType.DMA,
    ],
)
def _gather3(sv_n, oj_n, sv_r, oj_r, node_t, rel_t,
             h_out, r_out, t_out,
             svn_v, ojn_v, svr_v, ojr_v, ring, stage, semb, sems):
    wid = lax.axis_index("s") * _NC + lax.axis_index("c")
    pltpu.sync_copy(sv_n.at[pl.ds(wid * _HN, _HN)], svn_v)
    pltpu.sync_copy(oj_n.at[pl.ds(wid * _HN, _HN)], ojn_v)
    pltpu.sync_copy(sv_r.at[pl.ds(wid * _HR, _HR)], svr_v)
    pltpu.sync_copy(oj_r.at[pl.ds(wid * _HR, _HR)], ojr_v)

    rows16 = lax.iota(jnp.int32, _L)

    def run_stream(tab, sv_buf, oj_buf, nhits, out_a, out_b, split):
        b0 = lax.shift_right_logical(
            plsc.load_gather(sv_buf, [_splat(jnp.int32(0))])[0], 7)
        blast = lax.shift_right_logical(
            plsc.load_gather(sv_buf, [_splat(jnp.int32(nhits - 1))])[0], 7)
        nbt = blast - b0 + 1

        def issue_more(carry):
            drained, issued = carry

            def cond(c2):
                d2, i2 = c2
                return jnp.logical_and(i2 < nbt, i2 < d2 + _R - 1)

            def body(c2):
                d2, i2 = c2
                boff = pl.multiple_of((b0 + i2) * _BW, _BW)
                pltpu.async_copy(tab.at[:, pl.ds(boff, _BW)],
                                 ring.at[lax.rem(i2, _R)], semb)
                return (d2, i2 + 1)

            return lax.while_loop(cond, body, (drained, issued))

        def drain_to(needed, carry):
            drained, issued = carry

            def cond(c2):
                d2, i2 = c2
                return d2 < needed

            def body(c2):
                d2, i2 = c2
                pltpu.make_async_copy(tab.at[:, pl.ds(0, _BW)],
                                      ring.at[lax.rem(d2, _R)], semb).wait()
                return (d2 + 1, i2)

            return lax.while_loop(cond, body, (drained, issued))

        carry0 = issue_more((jnp.int32(0), jnp.int32(0)))

        def hit_body(i, carry):
            v = plsc.load_gather(sv_buf, [_splat(i)])[0]
            j = plsc.load_gather(oj_buf, [_splat(i)])[0]
            needed = lax.shift_right_logical(v, 7) - b0 + 1
            carry = drain_to(needed, carry)
            carry = issue_more(carry)
            slot = lax.rem(lax.shift_right_logical(v, 7) - b0, _R)
            col = lax.bitwise_and(v, jnp.int32(_BW - 1))
            st = lax.rem(i, _NST)

            @pl.when(i >= _NST)
            def _():
                pltpu.make_async_copy(stage.at[pl.ds(st, 1)],
                                      out_a.at[pl.ds(0, 1)], sems).wait()

            for q in range(_D // _L):
                rr = rows16 + q * _L
                x = plsc.load_gather(ring, [_splat(slot), rr, _splat(col)])
                plsc.store_scatter(stage, [_splat(st), rr], x)

            jj = jnp.where(j < _B, j, j - _B)

            @pl.when(j < split)
            def _():
                pltpu.async_copy(stage.at[pl.ds(st, 1)],
                                 out_a.at[pl.ds(jj, 1)], sems)

            @pl.when(j >= split)
            def _():
                pltpu.async_copy(stage.at[pl.ds(st, 1)],
                                 out_b.at[pl.ds(jj, 1)], sems)

            return carry

        carry1 = lax.fori_loop(0, nhits, hit_body, carry0)
        drained, issued = carry1
        _ = drain_to(issued, (drained, issued))
        nlast = jnp.minimum(jnp.int32(nhits), jnp.int32(_NST))

        def drain_stage(k, carry2):
            pltpu.make_async_copy(stage.at[pl.ds(lax.rem(k, _NST), 1)],
                                  out_a.at[pl.ds(0, 1)], sems).wait()
            return carry2

        lax.fori_loop(0, nlast, drain_stage, 0)

    run_stream(node_t, svn_v, ojn_v, _HN, h_out, t_out, jnp.int32(_B))
    run_stream(rel_t, svr_v, ojr_v, _HR, r_out, r_out, jnp.int32(_B))


def kernel(triplets, node_embs, rel_embs):
    tri = triplets.astype(jnp.int32)
    h_idx = tri[:, 0].reshape(_B)
    r_idx = tri[:, 1].reshape(_B)
    t_idx = tri[:, 2].reshape(_B)
    v_n = jnp.concatenate([h_idx, t_idx])
    order_n = jnp.argsort(v_n).astype(jnp.int32)
    sv_n = v_n[order_n]
    order_r = jnp.argsort(r_idx).astype(jnp.int32)
    sv_r = r_idx[order_r]
    head, rel, tail = _gather3(sv_n, order_n, sv_r, order_r,
                               node_embs.T, rel_embs.T)
    return (head, rel, tail)
